# baseline (speedup 1.0000x reference)
import functools

import jax
import jax.numpy as jnp
from jax import lax
from jax.experimental import pallas as pl
from jax.experimental.pallas import tpu as pltpu
from jax.experimental.pallas import tpu_sc as plsc

B = 16384
F = 16
NC = 2
NS = 16
NW = NC * NS
BPW = B // NW
CHUNK = 128
NCHUNK = BPW // CHUNK
GROUPS = BPW // 16


def _svdpp_body(uidx_hbm, iidx_hbm, ut_hbm, it_hbm, ubt_hbm, ibt_hbm,
                gb_hbm, out_hbm, uidx_v, iidx_v, urows_v, irows_v,
                ubias_v, ibias_v, out_v, gb_v, sem):
    wid = lax.axis_index("s") * NC + lax.axis_index("c")
    base = wid * NCHUNK

    pltpu.sync_copy(uidx_hbm.at[pl.ds(base, NCHUNK)], uidx_v)
    pltpu.sync_copy(iidx_hbm.at[pl.ds(base, NCHUNK)], iidx_v)
    pltpu.sync_copy(gb_hbm, gb_v)

    copies = []
    for j in range(NCHUNK):
        r = pl.ds(j * CHUNK, CHUNK)
        copies.append(pltpu.async_copy(ut_hbm.at[uidx_v.at[j]], urows_v.at[r], sem))
        copies.append(pltpu.async_copy(it_hbm.at[iidx_v.at[j]], irows_v.at[r], sem))
        copies.append(pltpu.async_copy(ubt_hbm.at[uidx_v.at[j]], ubias_v.at[r], sem))
        copies.append(pltpu.async_copy(ibt_hbm.at[iidx_v.at[j]], ibias_v.at[r], sem))
    for c in copies:
        c.wait()

    gb = gb_v[...]
    lane = lax.iota(jnp.int32, 16)

    def group(g, _):
        rows = g * 16 + lane
        acc = ubias_v[pl.ds(g * 16, 16)] + ibias_v[pl.ds(g * 16, 16)] + gb
        for f in range(F):
            col = jnp.full((16,), f, jnp.int32)
            u = plsc.load_gather(urows_v, [rows, col])
            v = plsc.load_gather(irows_v, [rows, col])
            acc = acc + u * v
        out_v[pl.ds(g * 16, 16)] = acc
        return _

    lax.fori_loop(0, GROUPS, group, 0)
    pltpu.sync_copy(out_v, out_hbm.at[pl.ds(wid * BPW, BPW)])


@jax.jit
def _svdpp(user_idx, item_idx, user_table, item_table,
           user_bias_table, item_bias_table, global_bias):
    mesh = plsc.VectorSubcoreMesh(core_axis_name="c", subcore_axis_name="s")
    kfn = functools.partial(
        pl.kernel,
        mesh=mesh,
        compiler_params=pltpu.CompilerParams(
            needs_layout_passes=False, use_tc_tiling_on_sc=False),
        out_type=jax.ShapeDtypeStruct((B,), jnp.float32),
        scratch_types=[
            pltpu.VMEM((NCHUNK, CHUNK), jnp.int32),
            pltpu.VMEM((NCHUNK, CHUNK), jnp.int32),
            pltpu.VMEM((BPW, F), jnp.float32),
            pltpu.VMEM((BPW, F), jnp.float32),
            pltpu.VMEM((BPW,), jnp.float32),
            pltpu.VMEM((BPW,), jnp.float32),
            pltpu.VMEM((BPW,), jnp.float32),
            pltpu.VMEM((16,), jnp.float32),
            pltpu.SemaphoreType.DMA,
        ],
    )(_svdpp_body)
    return kfn(user_idx.reshape(B // CHUNK, CHUNK),
               item_idx.reshape(B // CHUNK, CHUNK),
               user_table, item_table,
               user_bias_table.reshape(-1), item_bias_table.reshape(-1),
               jnp.broadcast_to(global_bias, (16,)))


def kernel(user_idx, item_idx, user_table, item_table, implicit_table,
           user_bias_table, item_bias_table, global_bias):
    del implicit_table
    return _svdpp(user_idx.astype(jnp.int32), item_idx.astype(jnp.int32),
                  user_table, item_table,
                  user_bias_table, item_bias_table, global_bias)
